# baseline (speedup 1.0000x reference)
import jax
import jax.numpy as jnp
from jax import lax
from jax.experimental import pallas as pl
from jax.experimental.pallas import tpu as pltpu
from jax.experimental.pallas import tpu_sc as plsc

N = 10000
E = 320000
R = 8
IN_DIM = 128
HID_DIM = 256
OUT_DIM = 128

NC = 2
NS = 16
CHW = 8
NCHUNK = IN_DIM // CHW
CPS = NCHUNK // NC

WSIZE = 256
GW = 256
SPG = GW // WSIZE
GRP = 8
NWIN = 80
NGW = 80
GGROUPS = NGW // GRP
EPT = NWIN * WSIZE
E_PAD = EPT * NS

SEG = N * R
ACC_PAD = 256
ACC_ROWS = SEG + ACC_PAD
ZERO_PT = ACC_ROWS // NS
OUT_PT = SEG // NS

BN = 1000
GRID = N // BN

_mesh = plsc.VectorSubcoreMesh(
    core_axis_name="c", subcore_axis_name="s", num_cores=NC, num_subcores=NS
)


def _edge_rounds(table, zeros_hbm, out_hbm,
                 gidx_v, sidx_v, rows_v, acc,
                 gsem0, gsem1, ssem0, ssem1, sid, cid):
    for k in range(CPS):
        chunk = cid * CPS + k
        pltpu.sync_copy(
            zeros_hbm.at[pl.ds(sid * ZERO_PT, ZERO_PT)],
            acc.at[pl.ds(sid * ZERO_PT, ZERO_PT)],
        )
        plsc.subcore_barrier()

        tk = table.at[chunk]

        def grp(g, carry):
            gd = []
            for b in range(GRP):
                w = g * GRP + b
                gd.append(pltpu.async_copy(tk.at[gidx_v.at[w]], rows_v.at[b], gsem0))
            for d in gd:
                d.wait()
            sd = []
            for b in range(GRP):
                w = g * GRP + b
                for hf in range(SPG):
                    sd.append(pltpu.async_copy(
                        rows_v.at[b, pl.ds(hf * WSIZE, WSIZE)],
                        acc.at[sidx_v.at[SPG * w + hf]], ssem0, add=True))
            for d in sd:
                d.wait()
            return carry

        lax.fori_loop(0, GGROUPS, grp, 0)
        plsc.subcore_barrier()
        pltpu.sync_copy(
            acc.at[pl.ds(sid * OUT_PT, OUT_PT)],
            out_hbm.at[pl.ds(sid * OUT_PT, OUT_PT), chunk],
        )
        plsc.subcore_barrier()


def _sc_layer1_body(table, gidx_hbm, sidx_hbm, zeros_hbm, ones_hbm,
                    out_hbm, counts_hbm,
                    gidx_v, sidx_v, rows_v, acc, gsem0, gsem1, ssem0, ssem1):
    cid = lax.axis_index("c")
    sid = lax.axis_index("s")
    pltpu.sync_copy(gidx_hbm.at[sid], gidx_v)
    pltpu.sync_copy(sidx_hbm.at[sid], sidx_v)
    _edge_rounds(table, zeros_hbm, out_hbm, gidx_v, sidx_v, rows_v, acc,
                 gsem0, gsem1, ssem0, ssem1, sid, cid)

    @pl.when(cid == 0)
    def _():
        pltpu.sync_copy(ones_hbm, rows_v.at[0, pl.ds(0, WSIZE)])
        pltpu.sync_copy(
            zeros_hbm.at[pl.ds(sid * ZERO_PT, ZERO_PT)],
            acc.at[pl.ds(sid * ZERO_PT, ZERO_PT)],
        )
        plsc.subcore_barrier()

        def cgrp(g, carry):
            for b in range(GRP):
                w = g * GRP + b
                pltpu.async_copy(
                    rows_v.at[0, pl.ds(0, WSIZE)], acc.at[sidx_v.at[w]],
                    ssem0, add=True,
                )
            for b in range(GRP):
                pltpu.make_async_copy(
                    rows_v.at[0, pl.ds(0, WSIZE)], acc.at[sidx_v.at[g * GRP + b]],
                    ssem0,
                ).wait()
            return carry

        lax.fori_loop(0, NWIN // GRP, cgrp, 0)
        plsc.subcore_barrier()
        pltpu.sync_copy(
            acc.at[pl.ds(sid * OUT_PT, OUT_PT)],
            counts_hbm.at[pl.ds(sid * OUT_PT, OUT_PT)],
        )


def _sc_layer2_body(table, gidx_hbm, sidx_hbm, zeros_hbm, out_hbm,
                    gidx_v, sidx_v, rows_v, acc, gsem0, gsem1, ssem0, ssem1):
    cid = lax.axis_index("c")
    sid = lax.axis_index("s")
    pltpu.sync_copy(gidx_hbm.at[sid], gidx_v)
    pltpu.sync_copy(sidx_hbm.at[sid], sidx_v)
    _edge_rounds(table, zeros_hbm, out_hbm, gidx_v, sidx_v, rows_v, acc,
                 gsem0, gsem1, ssem0, ssem1, sid, cid)


_SC_SCRATCH = [
    pltpu.VMEM((NGW, GW), jnp.int32),
    pltpu.VMEM((NWIN, WSIZE), jnp.int32),
    pltpu.VMEM((GRP, GW, CHW), jnp.float32),
    pltpu.VMEM_SHARED((ACC_ROWS, CHW), jnp.float32),
    pltpu.SemaphoreType.DMA,
    pltpu.SemaphoreType.DMA,
    pltpu.SemaphoreType.DMA,
    pltpu.SemaphoreType.DMA,
]

_SC_PARAMS = pltpu.CompilerParams(use_tc_tiling_on_sc=False)

_sc_layer1 = pl.kernel(
    _sc_layer1_body,
    out_type=(
        jax.ShapeDtypeStruct((SEG, NCHUNK, CHW), jnp.float32),
        jax.ShapeDtypeStruct((SEG, CHW), jnp.float32),
    ),
    mesh=_mesh,
    scratch_types=_SC_SCRATCH,
    compiler_params=_SC_PARAMS,
    name="rgcn_sc_layer1",
)

_sc_layer2 = pl.kernel(
    _sc_layer2_body,
    out_type=jax.ShapeDtypeStruct((SEG, NCHUNK, CHW), jnp.float32),
    mesh=_mesh,
    scratch_types=_SC_SCRATCH,
    compiler_params=_SC_PARAMS,
    name="rgcn_sc_layer2",
)


def _tc1_body(a1_ref, cnt_ref, x_ref, w1_ref, root1_ref, b1_ref, h_ref):
    inv = 1.0 / jnp.maximum(cnt_ref[...], 1.0)
    acc = jnp.dot(x_ref[...], root1_ref[...], preferred_element_type=jnp.float32)
    for r in range(R):
        a_r = a1_ref[:, r * IN_DIM:(r + 1) * IN_DIM]
        s_r = inv[:, r * CHW:r * CHW + 1]
        acc += jnp.dot(a_r * s_r, w1_ref[r], preferred_element_type=jnp.float32)
    h_ref[...] = jnp.maximum(acc + b1_ref[...], 0.0)


def _tc2_body(a2_ref, cnt_ref, h_ref, root2_ref, b2_ref, out_ref):
    inv = 1.0 / jnp.maximum(cnt_ref[...], 1.0)
    acc = jnp.dot(h_ref[...], root2_ref[...], preferred_element_type=jnp.float32)
    for r in range(R):
        a_r = a2_ref[:, r * OUT_DIM:(r + 1) * OUT_DIM]
        s_r = inv[:, r * CHW:r * CHW + 1]
        acc += a_r * s_r
    out_ref[...] = acc + b2_ref[...]


_tc_layer1 = pl.pallas_call(
    _tc1_body,
    grid=(GRID,),
    in_specs=[
        pl.BlockSpec((BN, R * IN_DIM), lambda i: (i, 0)),
        pl.BlockSpec((BN, R * CHW), lambda i: (i, 0)),
        pl.BlockSpec((BN, IN_DIM), lambda i: (i, 0)),
        pl.BlockSpec((R, IN_DIM, HID_DIM), lambda i: (0, 0, 0)),
        pl.BlockSpec((IN_DIM, HID_DIM), lambda i: (0, 0)),
        pl.BlockSpec((1, HID_DIM), lambda i: (0, 0)),
    ],
    out_specs=pl.BlockSpec((BN, HID_DIM), lambda i: (i, 0)),
    out_shape=jax.ShapeDtypeStruct((N, HID_DIM), jnp.float32),
)

CB = 4


def _tct2_body(h_ref, w_ref, t2_ref):
    for cc in range(CB):
        t2_ref[cc] = jnp.dot(h_ref[...], w_ref[cc], preferred_element_type=jnp.float32)


_tc_t2 = pl.pallas_call(
    _tct2_body,
    grid=(GRID, NCHUNK // CB),
    in_specs=[
        pl.BlockSpec((BN, HID_DIM), lambda i, c: (i, 0)),
        pl.BlockSpec((CB, HID_DIM, R * CHW), lambda i, c: (c, 0, 0)),
    ],
    out_specs=pl.BlockSpec((CB, BN, R * CHW), lambda i, c: (c, i, 0)),
    out_shape=jax.ShapeDtypeStruct((NCHUNK, N, R * CHW), jnp.float32),
)

_tc_layer2 = pl.pallas_call(
    _tc2_body,
    grid=(GRID,),
    in_specs=[
        pl.BlockSpec((BN, R * OUT_DIM), lambda i: (i, 0)),
        pl.BlockSpec((BN, R * CHW), lambda i: (i, 0)),
        pl.BlockSpec((BN, HID_DIM), lambda i: (i, 0)),
        pl.BlockSpec((HID_DIM, OUT_DIM), lambda i: (0, 0)),
        pl.BlockSpec((1, OUT_DIM), lambda i: (0, 0)),
    ],
    out_specs=pl.BlockSpec((BN, OUT_DIM), lambda i: (i, 0)),
    out_shape=jax.ShapeDtypeStruct((N, OUT_DIM), jnp.float32),
)


def kernel(x, edge_index, edge_type, W1, root1, b1, W2, root2, b2):
    src = edge_index[0]
    dst = edge_index[1]
    et = edge_type

    pad = E_PAD - E
    padr = jnp.arange(pad, dtype=jnp.int32)
    srcp = jnp.concatenate([src, padr % N])
    relp = jnp.concatenate([et, padr % R])
    sidx = jnp.concatenate([dst * R + et, SEG + padr % ACC_PAD]).astype(jnp.int32)
    gidx1 = srcp.reshape(NS, NGW, GW)
    gidx2 = (srcp * R + relp).reshape(NS, NGW, GW)
    sidx_r = sidx.reshape(NS, NWIN, WSIZE)

    zeros = jnp.zeros((ACC_ROWS, CHW), jnp.float32)
    ones = jnp.ones((WSIZE, CHW), jnp.float32)

    x_t = x.reshape(N, NCHUNK, CHW).transpose(1, 0, 2)

    a1raw, counts_raw = _sc_layer1(x_t, gidx1, sidx_r, zeros, ones)
    a1t = a1raw.reshape(N, R * IN_DIM)
    cnt = counts_raw.reshape(N, R * CHW)

    h = _tc_layer1(a1t, cnt, x, W1, root1, b1.reshape(1, HID_DIM))

    w2cs = W2.reshape(R, HID_DIM, NCHUNK, CHW).transpose(2, 1, 0, 3).reshape(
        NCHUNK, HID_DIM, R * CHW)
    t2t = _tc_t2(h, w2cs).reshape(NCHUNK, N * R, CHW)
    a2raw = _sc_layer2(t2t, gidx2, sidx_r, zeros)
    a2t = a2raw.reshape(N, R * OUT_DIM)

    return _tc_layer2(a2t, cnt, h, root2, b2.reshape(1, OUT_DIM))


# speedup vs baseline: 2.1849x; 2.1849x over previous
import jax
import jax.numpy as jnp
from jax import lax
from jax.experimental import pallas as pl
from jax.experimental.pallas import tpu as pltpu
from jax.experimental.pallas import tpu_sc as plsc

N = 10000
E = 320000
R = 8
IN_DIM = 128
HID_DIM = 256
OUT_DIM = 128

NC = 2
NS = 16
CHW = 16
NCHUNK = IN_DIM // CHW
CPS = NCHUNK // NC

WSIZE = 256
GW = 256
SPG = GW // WSIZE
GRP = 8
NWIN = 80
NGW = 80
GGROUPS = NGW // GRP
EPT = NWIN * WSIZE
E_PAD = EPT * NS

SEG = N * R
ACC_PAD = 256
ACC_ROWS = SEG + ACC_PAD
ZERO_PT = ACC_ROWS // NS
OUT_PT = SEG // NS

BN = 2000
GRID = N // BN

_mesh = plsc.VectorSubcoreMesh(
    core_axis_name="c", subcore_axis_name="s", num_cores=NC, num_subcores=NS
)


def _edge_rounds(table, zeros_hbm, out_hbm,
                 gidx_v, sidx_v, rows_v, acc,
                 gsem0, gsem1, ssem0, ssem1, sid, cid):
    for k in range(CPS):
        chunk = cid * CPS + k
        pltpu.sync_copy(
            zeros_hbm.at[pl.ds(sid * ZERO_PT, ZERO_PT)],
            acc.at[pl.ds(sid * ZERO_PT, ZERO_PT)],
        )
        plsc.subcore_barrier()

        tk = table.at[chunk]

        def grp(g, carry):
            gd = []
            for b in range(GRP):
                w = g * GRP + b
                gd.append(pltpu.async_copy(tk.at[gidx_v.at[w]], rows_v.at[b], gsem0))
            for d in gd:
                d.wait()
            sd = []
            for b in range(GRP):
                w = g * GRP + b
                for hf in range(SPG):
                    sd.append(pltpu.async_copy(
                        rows_v.at[b, pl.ds(hf * WSIZE, WSIZE)],
                        acc.at[sidx_v.at[SPG * w + hf]], ssem0, add=True))
            for d in sd:
                d.wait()
            return carry

        lax.fori_loop(0, GGROUPS, grp, 0)
        plsc.subcore_barrier()
        pltpu.sync_copy(
            acc.at[pl.ds(sid * OUT_PT, OUT_PT)],
            out_hbm.at[pl.ds(sid * OUT_PT, OUT_PT), chunk],
        )
        plsc.subcore_barrier()


def _sc_layer1_body(table, gidx_hbm, sidx_hbm, zeros_hbm, ones_hbm,
                    out_hbm, counts_hbm,
                    gidx_v, sidx_v, rows_v, acc, gsem0, gsem1, ssem0, ssem1):
    cid = lax.axis_index("c")
    sid = lax.axis_index("s")
    pltpu.sync_copy(gidx_hbm.at[sid], gidx_v)
    pltpu.sync_copy(sidx_hbm.at[sid], sidx_v)
    _edge_rounds(table, zeros_hbm, out_hbm, gidx_v, sidx_v, rows_v, acc,
                 gsem0, gsem1, ssem0, ssem1, sid, cid)

    @pl.when(cid == 0)
    def _():
        pltpu.sync_copy(ones_hbm, rows_v.at[0, pl.ds(0, WSIZE)])
        pltpu.sync_copy(
            zeros_hbm.at[pl.ds(sid * ZERO_PT, ZERO_PT)],
            acc.at[pl.ds(sid * ZERO_PT, ZERO_PT)],
        )
        plsc.subcore_barrier()

        def cgrp(g, carry):
            for b in range(GRP):
                w = g * GRP + b
                pltpu.async_copy(
                    rows_v.at[0, pl.ds(0, WSIZE)], acc.at[sidx_v.at[w]],
                    ssem0, add=True,
                )
            for b in range(GRP):
                pltpu.make_async_copy(
                    rows_v.at[0, pl.ds(0, WSIZE)], acc.at[sidx_v.at[g * GRP + b]],
                    ssem0,
                ).wait()
            return carry

        lax.fori_loop(0, NWIN // GRP, cgrp, 0)
        plsc.subcore_barrier()
        pltpu.sync_copy(
            acc.at[pl.ds(sid * OUT_PT, OUT_PT)],
            counts_hbm.at[pl.ds(sid * OUT_PT, OUT_PT)],
        )


def _sc_layer2_body(table, gidx_hbm, sidx_hbm, zeros_hbm, out_hbm,
                    gidx_v, sidx_v, rows_v, acc, gsem0, gsem1, ssem0, ssem1):
    cid = lax.axis_index("c")
    sid = lax.axis_index("s")
    pltpu.sync_copy(gidx_hbm.at[sid], gidx_v)
    pltpu.sync_copy(sidx_hbm.at[sid], sidx_v)
    _edge_rounds(table, zeros_hbm, out_hbm, gidx_v, sidx_v, rows_v, acc,
                 gsem0, gsem1, ssem0, ssem1, sid, cid)


_SC_SCRATCH = [
    pltpu.VMEM((NGW, GW), jnp.int32),
    pltpu.VMEM((NWIN, WSIZE), jnp.int32),
    pltpu.VMEM((GRP, GW, CHW), jnp.bfloat16),
    pltpu.VMEM_SHARED((ACC_ROWS, CHW), jnp.bfloat16),
    pltpu.SemaphoreType.DMA,
    pltpu.SemaphoreType.DMA,
    pltpu.SemaphoreType.DMA,
    pltpu.SemaphoreType.DMA,
]

_SC_PARAMS = pltpu.CompilerParams(use_tc_tiling_on_sc=False)

_sc_layer1 = pl.kernel(
    _sc_layer1_body,
    out_type=(
        jax.ShapeDtypeStruct((SEG, NCHUNK, CHW), jnp.bfloat16),
        jax.ShapeDtypeStruct((SEG, CHW), jnp.bfloat16),
    ),
    mesh=_mesh,
    scratch_types=_SC_SCRATCH,
    compiler_params=_SC_PARAMS,
    name="rgcn_sc_layer1",
)

_sc_layer2 = pl.kernel(
    _sc_layer2_body,
    out_type=jax.ShapeDtypeStruct((SEG, NCHUNK, CHW), jnp.bfloat16),
    mesh=_mesh,
    scratch_types=_SC_SCRATCH,
    compiler_params=_SC_PARAMS,
    name="rgcn_sc_layer2",
)


def _tc1_body(a1_ref, cnt_ref, x_ref, w1_ref, root1_ref, b1_ref, h_ref):
    inv = 1.0 / jnp.maximum(cnt_ref[...].astype(jnp.float32), 1.0)
    acc = jnp.dot(x_ref[...], root1_ref[...], preferred_element_type=jnp.float32)
    for r in range(R):
        a_r = a1_ref[:, r * IN_DIM:(r + 1) * IN_DIM].astype(jnp.float32)
        s_r = inv[:, r * CHW:r * CHW + 1]
        acc += jnp.dot(a_r * s_r, w1_ref[r], preferred_element_type=jnp.float32)
    h_ref[...] = jnp.maximum(acc + b1_ref[...], 0.0)


def _tc2_body(a2_ref, cnt_ref, h_ref, root2_ref, b2_ref, out_ref):
    inv = 1.0 / jnp.maximum(cnt_ref[...].astype(jnp.float32), 1.0)
    acc = jnp.dot(h_ref[...], root2_ref[...], preferred_element_type=jnp.float32)
    for r in range(R):
        a_r = a2_ref[:, r * OUT_DIM:(r + 1) * OUT_DIM].astype(jnp.float32)
        s_r = inv[:, r * CHW:r * CHW + 1]
        acc += a_r * s_r
    out_ref[...] = acc + b2_ref[...]


_tc_layer1 = pl.pallas_call(
    _tc1_body,
    grid=(GRID,),
    in_specs=[
        pl.BlockSpec((BN, R * IN_DIM), lambda i: (i, 0)),
        pl.BlockSpec((BN, R * CHW), lambda i: (i, 0)),
        pl.BlockSpec((BN, IN_DIM), lambda i: (i, 0)),
        pl.BlockSpec((R, IN_DIM, HID_DIM), lambda i: (0, 0, 0)),
        pl.BlockSpec((IN_DIM, HID_DIM), lambda i: (0, 0)),
        pl.BlockSpec((1, HID_DIM), lambda i: (0, 0)),
    ],
    out_specs=pl.BlockSpec((BN, HID_DIM), lambda i: (i, 0)),
    out_shape=jax.ShapeDtypeStruct((N, HID_DIM), jnp.float32),
)

CB = 4


def _tct2_body(h_ref, w_ref, t2_ref):
    for cc in range(CB):
        t2_ref[cc] = jnp.dot(
            h_ref[...], w_ref[cc], preferred_element_type=jnp.float32
        ).astype(jnp.bfloat16)


_tc_t2 = pl.pallas_call(
    _tct2_body,
    grid=(GRID, NCHUNK // CB),
    in_specs=[
        pl.BlockSpec((BN, HID_DIM), lambda i, c: (i, 0)),
        pl.BlockSpec((CB, HID_DIM, R * CHW), lambda i, c: (c, 0, 0)),
    ],
    out_specs=pl.BlockSpec((CB, BN, R * CHW), lambda i, c: (c, i, 0)),
    out_shape=jax.ShapeDtypeStruct((NCHUNK, N, R * CHW), jnp.bfloat16),
)

_tc_layer2 = pl.pallas_call(
    _tc2_body,
    grid=(GRID,),
    in_specs=[
        pl.BlockSpec((BN, R * OUT_DIM), lambda i: (i, 0)),
        pl.BlockSpec((BN, R * CHW), lambda i: (i, 0)),
        pl.BlockSpec((BN, HID_DIM), lambda i: (i, 0)),
        pl.BlockSpec((HID_DIM, OUT_DIM), lambda i: (0, 0)),
        pl.BlockSpec((1, OUT_DIM), lambda i: (0, 0)),
    ],
    out_specs=pl.BlockSpec((BN, OUT_DIM), lambda i: (i, 0)),
    out_shape=jax.ShapeDtypeStruct((N, OUT_DIM), jnp.float32),
)


def kernel(x, edge_index, edge_type, W1, root1, b1, W2, root2, b2):
    src = edge_index[0]
    dst = edge_index[1]
    et = edge_type

    pad = E_PAD - E
    padr = jnp.arange(pad, dtype=jnp.int32)
    srcp = jnp.concatenate([src, padr % N])
    relp = jnp.concatenate([et, padr % R])
    sidx = jnp.concatenate([dst * R + et, SEG + padr % ACC_PAD]).astype(jnp.int32)
    gidx1 = srcp.reshape(NS, NGW, GW)
    gidx2 = (srcp * R + relp).reshape(NS, NGW, GW)
    sidx_r = sidx.reshape(NS, NWIN, WSIZE)

    zeros = jnp.zeros((ACC_ROWS, CHW), jnp.bfloat16)
    ones = jnp.ones((WSIZE, CHW), jnp.bfloat16)

    x_t = x.astype(jnp.bfloat16).reshape(N, NCHUNK, CHW).transpose(1, 0, 2)

    a1raw, counts_raw = _sc_layer1(x_t, gidx1, sidx_r, zeros, ones)
    a1t = a1raw.reshape(N, R * IN_DIM)
    cnt = counts_raw.reshape(N, R * CHW)

    h = _tc_layer1(a1t, cnt, x, W1, root1, b1.reshape(1, HID_DIM))

    w2cs = W2.reshape(R, HID_DIM, NCHUNK, CHW).transpose(2, 1, 0, 3).reshape(
        NCHUNK, HID_DIM, R * CHW)
    t2t = _tc_t2(h, w2cs).reshape(NCHUNK, N * R, CHW)
    a2raw = _sc_layer2(t2t, gidx2, sidx_r, zeros)
    a2t = a2raw.reshape(N, R * OUT_DIM)

    return _tc_layer2(a2t, cnt, h, root2, b2.reshape(1, OUT_DIM))
